# baseline (speedup 1.0000x reference)
import functools

import jax
import jax.numpy as jnp
from jax import lax
from jax.experimental import pallas as pl
from jax.experimental.pallas import tpu as pltpu
from jax.experimental.pallas import tpu_sc as plsc

_L = 128
_NBUF = 4
_TILE = 64 * _L


def _gather_kernel(S, V, D, B0, NC, nb):
    mesh = plsc.VectorSubcoreMesh(core_axis_name="c", subcore_axis_name="s")
    NBLK = B0 // _L

    @functools.partial(
        pl.kernel,
        mesh=mesh,
        compiler_params=pltpu.CompilerParams(
            use_tc_tiling_on_sc=False, needs_layout_passes=False
        ),
        out_type=jax.ShapeDtypeStruct((S * NBLK * _TILE,), jnp.float32),
        scratch_types=[
            pltpu.VMEM((S, _L), jnp.int32),
            pltpu.VMEM((_NBUF, _L, D), jnp.float32),
            pltpu.VMEM((2 * _TILE,), jnp.float32),
            pltpu.SemaphoreType.DMA,
            pltpu.SemaphoreType.DMA,
        ],
    )
    def k(xT_hbm, tab_hbm, out_hbm, idx_v, rows_v, outb_v, gsem, wsem):
        wid = lax.axis_index("s") * NC + lax.axis_index("c")
        lane = lax.iota(jnp.int32, 16)
        colvecs = [
            [
                ((16 * j + lane) // 8) * (8 * _L)
                + ((16 * j + lane) % 8) * _L
                + ob * _TILE
                for j in range(D // 16)
            ]
            for ob in range(2)
        ]

        def fire_gather(s, slot):
            pltpu.async_copy(tab_hbm.at[idx_v.at[s]], rows_v.at[slot], gsem)

        def drain_gather(s, slot):
            pltpu.make_async_copy(
                tab_hbm.at[idx_v.at[s]], rows_v.at[slot], gsem
            ).wait()

        def shuffle(s, slot, oslot):
            rows = rows_v.at[slot]
            cvs = colvecs[oslot]

            def lgbody(lg, _):
                l0 = lg * 16
                for li in range(16):
                    l = l0 + li
                    for j in range(D // 16):
                        val = rows[l, pl.ds(16 * j, 16)]
                        plsc.store_scatter(outb_v, [cvs[j] + l], val)
                return 0

            lax.fori_loop(0, _L // 16, lgbody, 0)

        def write_tiles(s, oslot, blk, wait):
            base = (s * NBLK + blk) * (8 * _L)
            for a in range(D // 8):
                src = outb_v.at[pl.ds(oslot * _TILE + a * (8 * _L), 8 * _L)]
                dst = out_hbm.at[pl.ds(base + a * NBLK * (8 * _L), 8 * _L)]
                if wait:
                    pltpu.make_async_copy(src, dst, wsem).wait()
                else:
                    pltpu.async_copy(src, dst, wsem)

        def step(s, slot, oslot, blk):
            @pl.when(s >= 2)
            def _():
                write_tiles(s - 2, oslot, blk, wait=True)

            @pl.when(s + _NBUF - 1 < S)
            def _():
                fire_gather(s + _NBUF - 1, (slot + _NBUF - 1) % _NBUF)

            drain_gather(s, slot)
            shuffle(s, slot, oslot)
            write_tiles(s, oslot, blk, wait=False)

        def bibody(bi, _):
            blk = wid * nb + bi
            bcol = pl.multiple_of(blk * _L, _L)
            pltpu.sync_copy(xT_hbm.at[:, pl.ds(bcol, _L)], idx_v)

            for s in range(_NBUF - 1):
                fire_gather(s, s % _NBUF)

            def body(h, _):
                s0 = h * _NBUF
                for j in range(_NBUF):
                    step(s0 + j, j, j % 2, blk)
                return 0

            n_full = (S - 2) // _NBUF
            lax.fori_loop(0, n_full, body, 0)
            step(S - 2, (S - 2) % _NBUF, 0, blk)
            step(S - 1, (S - 1) % _NBUF, 1, blk)
            write_tiles(S - 2, 0, blk, wait=True)
            write_tiles(S - 1, 1, blk, wait=True)
            return 0

        lax.fori_loop(0, nb, bibody, 0)

    return k


def kernel(x, table):
    B0, S = x.shape
    V, D = table.shape

    info = plsc.get_sparse_core_info()
    NC, NS = info.num_cores, info.num_subcores
    NW = NC * NS
    assert B0 % (NW * _L) == 0
    nb = B0 // (NW * _L)
    assert (S - 2) % _NBUF == 0

    xT = x.T.astype(jnp.int32)
    k = _gather_kernel(S, V, D, B0, NC, nb)
    out_flat = k(xT, table)
    out5 = out_flat.reshape(S, D // 8, B0 // _L, 8, _L)
    return out5.transpose(2, 4, 0, 1, 3).reshape(B0, S, D)


# speedup vs baseline: 1.5124x; 1.5124x over previous
import functools

import jax
import jax.numpy as jnp
from jax import lax
from jax.experimental import pallas as pl
from jax.experimental.pallas import tpu as pltpu
from jax.experimental.pallas import tpu_sc as plsc

_L = 128
_LP = _L + 1
_NBUF = 4


def _gather_kernel(S, V, D, B0, NC, nb):
    mesh = plsc.VectorSubcoreMesh(core_axis_name="c", subcore_axis_name="s")
    NBLK = B0 // _L
    TA, TR = D // 8, 8

    @functools.partial(
        pl.kernel,
        mesh=mesh,
        compiler_params=pltpu.CompilerParams(
            use_tc_tiling_on_sc=False, needs_layout_passes=False
        ),
        out_type=jax.ShapeDtypeStruct((S, TA, NBLK, TR, _L), jnp.float32),
        scratch_types=[
            pltpu.VMEM((S, _L), jnp.int32),
            pltpu.VMEM((_NBUF, _L, D), jnp.float32),
            pltpu.VMEM((2, TA, TR, _LP), jnp.float32),
            pltpu.SemaphoreType.DMA,
            pltpu.SemaphoreType.DMA,
        ],
    )
    def k(xT_hbm, tab_hbm, out_hbm, idx_v, rows_v, outb_v, gsem, wsem):
        wid = lax.axis_index("s") * NC + lax.axis_index("c")
        lane = lax.iota(jnp.int32, 16)
        zero16 = lane - lane
        avecs = [(16 * j + lane) // TR for j in range(D // 16)]
        rvecs = [(16 * j + lane) % TR for j in range(D // 16)]
        obvecs = [zero16, zero16 + 1]

        def fire_gather(s, slot):
            pltpu.async_copy(tab_hbm.at[idx_v.at[s]], rows_v.at[slot], gsem)

        def drain_gather(s, slot):
            pltpu.make_async_copy(
                tab_hbm.at[idx_v.at[s]], rows_v.at[slot], gsem
            ).wait()

        def shuffle(s, slot, oslot):
            rows = rows_v.at[slot]
            obv = obvecs[oslot]

            def lgbody(lg, _):
                l0 = lg * 16
                for li in range(16):
                    l = l0 + li
                    lvec = zero16 + l
                    for j in range(D // 16):
                        val = rows[l, pl.ds(16 * j, 16)]
                        plsc.store_scatter(
                            outb_v, [obv, avecs[j], rvecs[j], lvec], val
                        )
                return 0

            lax.fori_loop(0, _L // 16, lgbody, 0)

        def write_tiles(s, oslot, blk, wait):
            for a in range(TA):
                src = outb_v.at[oslot, a, :, pl.ds(0, _L)]
                dst = out_hbm.at[s, a, blk, :, :]
                if wait:
                    pltpu.make_async_copy(src, dst, wsem).wait()
                else:
                    pltpu.async_copy(src, dst, wsem)

        def step(s, slot, oslot, blk):
            @pl.when(s >= 2)
            def _():
                write_tiles(s - 2, oslot, blk, wait=True)

            @pl.when(s + _NBUF - 1 < S)
            def _():
                fire_gather(s + _NBUF - 1, (slot + _NBUF - 1) % _NBUF)

            drain_gather(s, slot)
            shuffle(s, slot, oslot)
            write_tiles(s, oslot, blk, wait=False)

        def bibody(bi, _):
            blk = wid * nb + bi
            bcol = pl.multiple_of(blk * _L, _L)
            pltpu.sync_copy(xT_hbm.at[:, pl.ds(bcol, _L)], idx_v)

            for s in range(_NBUF - 1):
                fire_gather(s, s % _NBUF)

            def body(h, _):
                s0 = h * _NBUF
                for j in range(_NBUF):
                    step(s0 + j, j, j % 2, blk)
                return 0

            n_full = (S - 2) // _NBUF
            lax.fori_loop(0, n_full, body, 0)
            step(S - 2, (S - 2) % _NBUF, 0, blk)
            step(S - 1, (S - 1) % _NBUF, 1, blk)
            write_tiles(S - 2, 0, blk, wait=True)
            write_tiles(S - 1, 1, blk, wait=True)
            return 0

        lax.fori_loop(0, nb, bibody, 0)

    return k


def kernel(x, table):
    B0, S = x.shape
    V, D = table.shape

    info = plsc.get_sparse_core_info()
    NC, NS = info.num_cores, info.num_subcores
    NW = NC * NS
    assert B0 % (NW * _L) == 0
    nb = B0 // (NW * _L)
    assert (S - 2) % _NBUF == 0

    xT = x.T.astype(jnp.int32)
    k = _gather_kernel(S, V, D, B0, NC, nb)
    out5 = k(xT, table)
    return out5.transpose(2, 4, 0, 1, 3).reshape(B0, S, D)
